# baseline (speedup 1.0000x reference)
import functools

import jax
import jax.numpy as jnp
from jax import lax
from jax.experimental import pallas as pl
from jax.experimental.pallas import tpu as pltpu
from jax.experimental.pallas import tpu_sc as plsc

B = 16384
D = 64
DP = 128
NUM_ROWS = 1000000
NPACK = NUM_ROWS // 2
NUM_CORES = 2
NUM_SUBCORES = 16
NW = NUM_CORES * NUM_SUBCORES
BPW = B // NW
L = 16
RC = 320
NCHUNKS = NUM_ROWS // RC
C2 = 128
NC2 = BPW // C2


@functools.partial(
    pl.kernel,
    out_type=(
        jax.ShapeDtypeStruct((NPACK, DP), jnp.float32),
        jax.ShapeDtypeStruct((NPACK, DP), jnp.float32),
    ),
    mesh=plsc.VectorSubcoreMesh(core_axis_name="c", subcore_axis_name="s"),
    scratch_types=[
        pltpu.VMEM((RC // 8, 8, D), jnp.float32),
        pltpu.VMEM((RC // 8, 8, D), jnp.float32),
        pltpu.VMEM((RC // 2, DP), jnp.float32),
        pltpu.SemaphoreType.DMA,
        pltpu.SemaphoreType.DMA,
        pltpu.SemaphoreType.DMA,
    ],
    compiler_params=pltpu.CompilerParams(needs_layout_passes=False),
)
def _repack(eu_h, ei_h, ou_h, oi_h, in0, in1, buf_out, ra, rb, ws):
    wid = lax.axis_index("s") * NUM_CORES + lax.axis_index("c")
    n_my = (NCHUNKS - wid + NW - 1) // NW

    def pack_and_write(src_buf, dst_h, chunk):
        def pack8(p8, carry2):
            p0 = p8 * 8
            for dp in range(8):
                p = p0 + dp
                r0, r1 = 2 * p, 2 * p + 1
                for h in range(4):
                    buf_out[p, pl.ds(h * L, L)] = (
                        src_buf[r0 // 8, r0 % 8, pl.ds(h * L, L)])
                    buf_out[p, pl.ds(D + h * L, L)] = (
                        src_buf[r1 // 8, r1 % 8, pl.ds(h * L, L)])
            return carry2

        lax.fori_loop(0, RC // 16, pack8, 0)
        pltpu.async_copy(
            buf_out, dst_h.at[pl.ds(chunk * (RC // 2), RC // 2)], ws).wait()

    def run_table(src_h, dst_h):
        pltpu.async_copy(src_h.at[pl.ds(wid * (RC // 8), RC // 8)], in0, ra)

        def body(k2, carry):
            k = 2 * k2
            chunk = wid + k * NW

            @pl.when(k + 1 < n_my)
            def _():
                pltpu.async_copy(
                    src_h.at[pl.ds((chunk + NW) * (RC // 8), RC // 8)],
                    in1, rb)

            pltpu.make_async_copy(
                src_h.at[pl.ds(0, RC // 8)], in0, ra).wait()
            pack_and_write(in0, dst_h, chunk)

            @pl.when(k + 2 < n_my)
            def _():
                pltpu.async_copy(
                    src_h.at[pl.ds((chunk + 2 * NW) * (RC // 8), RC // 8)],
                    in0, ra)

            @pl.when(k + 1 < n_my)
            def _():
                pltpu.make_async_copy(
                    src_h.at[pl.ds(0, RC // 8)], in1, rb).wait()
                pack_and_write(in1, dst_h, chunk + NW)

            return carry

        lax.fori_loop(0, (n_my + 1) // 2, body, 0)

    run_table(eu_h, ou_h)
    run_table(ei_h, oi_h)


@functools.partial(
    pl.kernel,
    out_type=(
        jax.ShapeDtypeStruct((B,), jnp.float32),
        jax.ShapeDtypeStruct((B,), jnp.float32),
    ),
    mesh=plsc.VectorSubcoreMesh(core_axis_name="c", subcore_axis_name="s"),
    scratch_types=[
        pltpu.VMEM((NC2, C2), jnp.int32),
        pltpu.VMEM((NC2, C2), jnp.int32),
        pltpu.VMEM((NC2, C2), jnp.int32),
        pltpu.VMEM((NC2, C2), jnp.int32),
        pltpu.VMEM((NC2, C2), jnp.int32),
        pltpu.VMEM((NC2, C2), jnp.int32),
        pltpu.VMEM((C2, DP), jnp.float32),
        pltpu.VMEM((C2, DP), jnp.float32),
        pltpu.VMEM((C2, DP), jnp.float32),
        pltpu.VMEM((BPW,), jnp.float32),
        pltpu.VMEM((BPW,), jnp.float32),
        pltpu.SemaphoreType.DMA,
    ],
    compiler_params=pltpu.CompilerParams(needs_layout_passes=False),
)
def _gather_dot(user_h, item_i_h, item_j_h, eu_h, ei_h, oi_h, oj_h,
                idx_u, idx_i, idx_j, pid_u, pid_i, pid_j,
                u_v, vi_v, vj_v, oi_v, oj_v, sem):
    wid = lax.axis_index("s") * NUM_CORES + lax.axis_index("c")
    base = wid * BPW
    for c in range(NC2):
        pltpu.sync_copy(user_h.at[pl.ds(base + c * C2, C2)], idx_u.at[c])
        pltpu.sync_copy(item_i_h.at[pl.ds(base + c * C2, C2)], idx_i.at[c])
        pltpu.sync_copy(item_j_h.at[pl.ds(base + c * C2, C2)], idx_j.at[c])
    for c in range(NC2):
        for g in range(C2 // L):
            s = pl.ds(g * L, L)
            pid_u[c, s] = idx_u[c, s] >> 1
            pid_i[c, s] = idx_i[c, s] >> 1
            pid_j[c, s] = idx_j[c, s] >> 1

    lanes = lax.iota(jnp.int32, L)

    def chunk_body(c, carry):
        cu = pltpu.async_copy(eu_h.at[pid_u.at[c]], u_v, sem)
        ci = pltpu.async_copy(ei_h.at[pid_i.at[c]], vi_v, sem)
        cj = pltpu.async_copy(ei_h.at[pid_j.at[c]], vj_v, sem)
        cu.wait()
        ci.wait()
        cj.wait()
        for g in range(C2 // L):
            s = pl.ds(g * L, L)
            rows = g * L + lanes
            off_u = (idx_u[c, s] & 1) * D
            off_i = (idx_i[c, s] & 1) * D
            off_j = (idx_j[c, s] & 1) * D
            acc_i = jnp.zeros((L,), jnp.float32)
            acc_j = jnp.zeros((L,), jnp.float32)
            for k in range(D):
                u = plsc.load_gather(u_v, [rows, off_u + k])
                vi = plsc.load_gather(vi_v, [rows, off_i + k])
                vj = plsc.load_gather(vj_v, [rows, off_j + k])
                acc_i = acc_i + u * vi
                acc_j = acc_j + u * vj
            oi_v[pl.ds(c * C2 + g * L, L)] = acc_i
            oj_v[pl.ds(c * C2 + g * L, L)] = acc_j
        return carry

    lax.fori_loop(0, NC2, chunk_body, 0)

    pltpu.sync_copy(oi_v, oi_h.at[pl.ds(base, BPW)])
    pltpu.sync_copy(oj_v, oj_h.at[pl.ds(base, BPW)])


def kernel(user, item_i, item_j, embed_user, embed_item):
    eu3 = embed_user.reshape(NUM_ROWS // 8, 8, D)
    ei3 = embed_item.reshape(NUM_ROWS // 8, 8, D)
    eu_p, ei_p = _repack(eu3, ei3)
    return _gather_dot(user.astype(jnp.int32), item_i.astype(jnp.int32),
                       item_j.astype(jnp.int32), eu_p, ei_p)


# speedup vs baseline: 1.2878x; 1.2878x over previous
import functools

import jax
import jax.numpy as jnp
from jax import lax
from jax.experimental import pallas as pl
from jax.experimental.pallas import tpu as pltpu
from jax.experimental.pallas import tpu_sc as plsc

B = 16384
D = 64
TG = 8
NUM_ROWS = 1000000
NUM_CORES = 2
NUM_SUBCORES = 16
NW = NUM_CORES * NUM_SUBCORES
BPW = B // NW
L = 16
C2 = 64
NC2 = BPW // C2


@functools.partial(
    pl.kernel,
    out_type=(
        jax.ShapeDtypeStruct((B,), jnp.float32),
        jax.ShapeDtypeStruct((B,), jnp.float32),
    ),
    mesh=plsc.VectorSubcoreMesh(core_axis_name="c", subcore_axis_name="s"),
    scratch_types=[
        pltpu.VMEM((NC2, C2), jnp.int32),
        pltpu.VMEM((NC2, C2), jnp.int32),
        pltpu.VMEM((NC2, C2), jnp.int32),
        pltpu.VMEM((NC2, C2), jnp.int32),
        pltpu.VMEM((NC2, C2), jnp.int32),
        pltpu.VMEM((NC2, C2), jnp.int32),
        pltpu.VMEM((C2, TG, D), jnp.float32),
        pltpu.VMEM((C2, TG, D), jnp.float32),
        pltpu.VMEM((C2, TG, D), jnp.float32),
        pltpu.VMEM((BPW,), jnp.float32),
        pltpu.VMEM((BPW,), jnp.float32),
        pltpu.SemaphoreType.DMA,
    ],
    compiler_params=pltpu.CompilerParams(
        needs_layout_passes=False, use_tc_tiling_on_sc=False),
)
def _gather_dot(user_h, item_i_h, item_j_h, eu_h, ei_h, oi_h, oj_h,
                idx_u, idx_i, idx_j, gid_u, gid_i, gid_j,
                u_v, vi_v, vj_v, oi_v, oj_v, sem):
    wid = lax.axis_index("s") * NUM_CORES + lax.axis_index("c")
    base = wid * BPW
    for c in range(NC2):
        pltpu.sync_copy(user_h.at[pl.ds(base + c * C2, C2)], idx_u.at[c])
        pltpu.sync_copy(item_i_h.at[pl.ds(base + c * C2, C2)], idx_i.at[c])
        pltpu.sync_copy(item_j_h.at[pl.ds(base + c * C2, C2)], idx_j.at[c])
    for c in range(NC2):
        for g in range(C2 // L):
            s = pl.ds(g * L, L)
            gid_u[c, s] = idx_u[c, s] >> 3
            gid_i[c, s] = idx_i[c, s] >> 3
            gid_j[c, s] = idx_j[c, s] >> 3

    lanes = lax.iota(jnp.int32, L)

    def chunk_body(c, carry):
        cu = pltpu.async_copy(eu_h.at[gid_u.at[c]], u_v, sem)
        ci = pltpu.async_copy(ei_h.at[gid_i.at[c]], vi_v, sem)
        cj = pltpu.async_copy(ei_h.at[gid_j.at[c]], vj_v, sem)
        cu.wait()
        ci.wait()
        cj.wait()
        for g in range(C2 // L):
            s = pl.ds(g * L, L)
            slots = g * L + lanes
            ru = idx_u[c, s] & 7
            ri = idx_i[c, s] & 7
            rj = idx_j[c, s] & 7
            acc_i = jnp.zeros((L,), jnp.float32)
            acc_j = jnp.zeros((L,), jnp.float32)
            for k in range(D):
                col = jnp.full((L,), k, dtype=jnp.int32)
                u = plsc.load_gather(u_v, [slots, ru, col])
                vi = plsc.load_gather(vi_v, [slots, ri, col])
                vj = plsc.load_gather(vj_v, [slots, rj, col])
                acc_i = acc_i + u * vi
                acc_j = acc_j + u * vj
            oi_v[pl.ds(c * C2 + g * L, L)] = acc_i
            oj_v[pl.ds(c * C2 + g * L, L)] = acc_j
        return carry

    lax.fori_loop(0, NC2, chunk_body, 0)

    pltpu.sync_copy(oi_v, oi_h.at[pl.ds(base, BPW)])
    pltpu.sync_copy(oj_v, oj_h.at[pl.ds(base, BPW)])


def kernel(user, item_i, item_j, embed_user, embed_item):
    eu3 = embed_user.reshape(NUM_ROWS // TG, TG, D)
    ei3 = embed_item.reshape(NUM_ROWS // TG, TG, D)
    return _gather_dot(user.astype(jnp.int32), item_i.astype(jnp.int32),
                       item_j.astype(jnp.int32), eu3, ei3)


# speedup vs baseline: 1.8751x; 1.4561x over previous
import functools

import jax
import jax.numpy as jnp
from jax import lax
from jax.experimental import pallas as pl
from jax.experimental.pallas import tpu as pltpu
from jax.experimental.pallas import tpu_sc as plsc

B = 16384
D = 64
NUM_CORES = 2
NUM_SUBCORES = 16
NW = NUM_CORES * NUM_SUBCORES
BPW = B // NW
L = 16
TG = 8


@functools.partial(
    pl.kernel,
    out_type=(
        jax.ShapeDtypeStruct((B,), jnp.float32),
        jax.ShapeDtypeStruct((B,), jnp.float32),
    ),
    mesh=plsc.VectorSubcoreMesh(core_axis_name="c", subcore_axis_name="s"),
    scratch_types=[
        pltpu.VMEM((BPW,), jnp.int32),
        pltpu.VMEM((BPW,), jnp.int32),
        pltpu.VMEM((BPW,), jnp.int32),
        pltpu.VMEM((L, TG, D), jnp.float32),
        pltpu.VMEM((L, TG, D), jnp.float32),
        pltpu.VMEM((L, TG, D), jnp.float32),
        pltpu.VMEM((BPW,), jnp.float32),
        pltpu.VMEM((BPW,), jnp.float32),
        pltpu.SemaphoreType.DMA,
    ],
    compiler_params=pltpu.CompilerParams(needs_layout_passes=False),
)
def _mfbpr(user_h, item_i_h, item_j_h, eu_h, ei_h, oi_h, oj_h,
           idx_u, idx_i, idx_j, tb_u, tb_i, tb_j, oi_v, oj_v, sem):
    wid = lax.axis_index("s") * NUM_CORES + lax.axis_index("c")
    base = wid * BPW
    pltpu.sync_copy(user_h.at[pl.ds(base, BPW)], idx_u)
    pltpu.sync_copy(item_i_h.at[pl.ds(base, BPW)], idx_i)
    pltpu.sync_copy(item_j_h.at[pl.ds(base, BPW)], idx_j)

    lanes = lax.iota(jnp.int32, L)

    def chunk_body(c, carry):
        rbase = c * L
        iu = idx_u[pl.ds(rbase, L)]
        ii = idx_i[pl.ds(rbase, L)]
        ij = idx_j[pl.ds(rbase, L)]
        copies = []
        for t in range(L):
            gu = (iu[t] >> 3) * TG
            gi = (ii[t] >> 3) * TG
            gj = (ij[t] >> 3) * TG
            copies.append(
                pltpu.async_copy(eu_h.at[pl.ds(gu, TG)], tb_u.at[t], sem))
            copies.append(
                pltpu.async_copy(ei_h.at[pl.ds(gi, TG)], tb_i.at[t], sem))
            copies.append(
                pltpu.async_copy(ei_h.at[pl.ds(gj, TG)], tb_j.at[t], sem))
        for cp in copies:
            cp.wait()
        ru = iu & 7
        ri = ii & 7
        rj = ij & 7
        acc_i = jnp.zeros((L,), jnp.float32)
        acc_j = jnp.zeros((L,), jnp.float32)
        for k in range(D):
            col = jnp.full((L,), k, dtype=jnp.int32)
            u = plsc.load_gather(tb_u, [lanes, ru, col])
            vi = plsc.load_gather(tb_i, [lanes, ri, col])
            vj = plsc.load_gather(tb_j, [lanes, rj, col])
            acc_i = acc_i + u * vi
            acc_j = acc_j + u * vj
        oi_v[pl.ds(rbase, L)] = acc_i
        oj_v[pl.ds(rbase, L)] = acc_j
        return carry

    lax.fori_loop(0, BPW // L, chunk_body, 0)

    pltpu.sync_copy(oi_v, oi_h.at[pl.ds(base, BPW)])
    pltpu.sync_copy(oj_v, oj_h.at[pl.ds(base, BPW)])


def kernel(user, item_i, item_j, embed_user, embed_item):
    return _mfbpr(user.astype(jnp.int32), item_i.astype(jnp.int32),
                  item_j.astype(jnp.int32), embed_user, embed_item)
